# baseline (speedup 1.0000x reference)
import functools

import jax
import jax.numpy as jnp
from jax import lax
from jax.experimental import pallas as pl
from jax.experimental.pallas import tpu as pltpu
from jax.experimental.pallas import tpu_sc as plsc

B = 1024
NPATHS = 200
D = 128
LABELS = 1000
NEG_INF = -2.0 * 10**10

NC = 2
NS = 16
NW = NC * NS

NSPLIT = 4
BC = B // NSPLIT
TOTC = BC * NPATHS
PER_W = TOTC // NW
CHUNK = 80
N_CHUNKS = PER_W // CHUNK


def _sc_gather_body(chunk, starts_h, paths_h, ends_h, vt_h, pt_h,
                    os_h, op_h, oe_h,
                    idx_s, idx_p, idx_e,
                    r00, r01, r02, r10, r11, r12,
                    gsem0, gsem1, ssem0, ssem1):
    wid = lax.axis_index("s") * NC + lax.axis_index("c")
    base = wid * PER_W
    gsem = (gsem0, gsem1)
    ssem = (ssem0, ssem1)
    tabs = (vt_h, pt_h, vt_h)
    outs = (os_h, op_h, oe_h)
    idxs = (idx_s, idx_p, idx_e)
    rows = ((r00, r01, r02), (r10, r11, r12))

    for t, idx_h in enumerate((starts_h, paths_h, ends_h)):
        pltpu.sync_copy(idx_h.at[pl.ds(chunk * TOTC + base, PER_W)], idxs[t])

    def issue_gathers(slot, g):
        for t in range(3):
            pltpu.async_copy(
                tabs[t].at[idxs[t].at[pl.ds(g * CHUNK, CHUNK)]],
                rows[slot][t], gsem[slot])

    def wait_gathers(slot):
        for t in range(3):
            pltpu.make_async_copy(
                tabs[t].at[pl.ds(0, CHUNK), :],
                rows[slot][t], gsem[slot]).wait()

    def issue_stores(slot, g):
        for t in range(3):
            pltpu.async_copy(
                rows[slot][t],
                outs[t].at[pl.ds(base + g * CHUNK, CHUNK), :], ssem[slot])

    def wait_stores(slot):
        for t in range(3):
            pltpu.make_async_copy(
                rows[slot][t],
                outs[t].at[pl.ds(0, CHUNK), :], ssem[slot]).wait()

    issue_gathers(0, 0)
    issue_gathers(1, 1)

    def body(j, _):
        g0 = 2 * j
        g1 = g0 + 1
        wait_gathers(0)
        issue_stores(0, g0)
        wait_gathers(1)
        issue_stores(1, g1)
        wait_stores(0)

        @pl.when(g0 + 2 < N_CHUNKS)
        def _():
            issue_gathers(0, g0 + 2)
        wait_stores(1)

        @pl.when(g1 + 2 < N_CHUNKS)
        def _():
            issue_gathers(1, g1 + 2)
        return 0

    lax.fori_loop(0, N_CHUNKS // 2, body, 0)


@functools.cache
def _sc_gather(chunk):
    return functools.partial(
        pl.kernel,
        mesh=plsc.VectorSubcoreMesh(core_axis_name="c", subcore_axis_name="s"),
        out_type=(
            jax.ShapeDtypeStruct((TOTC, D), jnp.float32),
            jax.ShapeDtypeStruct((TOTC, D), jnp.float32),
            jax.ShapeDtypeStruct((TOTC, D), jnp.float32),
        ),
        scratch_types=(
            [pltpu.VMEM((PER_W,), jnp.int32)] * 3
            + [pltpu.VMEM((CHUNK, D), jnp.float32)] * 6
            + [pltpu.SemaphoreType.DMA] * 4
        ),
    )(functools.partial(_sc_gather_body, chunk))


NRAW = 2


def _sc_gsum_body(chunk, starts_h, paths_h, ends_h, vws_h, pwp_h, vwe_h,
                  out_h,
                  idx_s, idx_p, idx_e,
                  g00, g01, g02, g10, g11, g12, sum0, sum1,
                  gsem0, gsem1, ssem0, ssem1):
    wid = lax.axis_index("s") * NC + lax.axis_index("c")
    base = wid * PER_W
    gsem = (gsem0, gsem1)
    ssem = (ssem0, ssem1)
    tabs = (vws_h, pwp_h, vwe_h)
    idxs = (idx_s, idx_p, idx_e)
    gbuf = ((g00, g01, g02), (g10, g11, g12))
    sbuf = (sum0, sum1)

    for t, idx_h in enumerate((starts_h, paths_h, ends_h)):
        pltpu.sync_copy(idx_h.at[pl.ds(chunk * TOTC + base, PER_W)], idxs[t])

    def issue_gathers(slot, g):
        for t in range(3):
            pltpu.async_copy(
                tabs[t].at[idxs[t].at[pl.ds(g * CHUNK, CHUNK)]],
                gbuf[slot][t], gsem[slot])

    def wait_gathers(slot):
        for t in range(3):
            pltpu.make_async_copy(
                tabs[t].at[pl.ds(0, CHUNK), :],
                gbuf[slot][t], gsem[slot]).wait()

    def compute_sum(slot):
        b0, b1, b2 = gbuf[slot]
        sb = sbuf[slot]

        def row_body(r, _):
            for l in range(D // 16):
                sl = pl.ds(l * 16, 16)
                sb[r, sl] = b0[r, sl] + b1[r, sl] + b2[r, sl]
            return 0
        lax.fori_loop(0, CHUNK, row_body, 0)

    def issue_store(slot, g):
        pltpu.async_copy(
            sbuf[slot], out_h.at[pl.ds(base + g * CHUNK, CHUNK), :],
            ssem[slot])

    def wait_store(slot):
        pltpu.make_async_copy(
            sbuf[slot], out_h.at[pl.ds(0, CHUNK), :], ssem[slot]).wait()

    issue_gathers(0, 0)

    def body(j, _):
        g0 = 2 * j
        g1 = g0 + 1
        wait_gathers(0)
        issue_gathers(1, g1)

        @pl.when(g0 >= 2)
        def _():
            wait_store(0)
        compute_sum(0)
        issue_store(0, g0)
        wait_gathers(1)

        @pl.when(g0 + 2 < N_CHUNKS)
        def _():
            issue_gathers(0, g0 + 2)

        @pl.when(g1 >= 2)
        def _():
            wait_store(1)
        compute_sum(1)
        issue_store(1, g1)
        return 0

    lax.fori_loop(0, N_CHUNKS // 2, body, 0)
    wait_store(0)
    wait_store(1)


@functools.cache
def _sc_gsum(chunk):
    return functools.partial(
        pl.kernel,
        mesh=plsc.VectorSubcoreMesh(core_axis_name="c", subcore_axis_name="s"),
        out_type=jax.ShapeDtypeStruct((TOTC, D), jnp.float32),
        scratch_types=(
            [pltpu.VMEM((PER_W,), jnp.int32)] * 3
            + [pltpu.VMEM((CHUNK, D), jnp.float32)] * 8
            + [pltpu.SemaphoreType.DMA] * 4
        ),
    )(functools.partial(_sc_gsum_body, chunk))


PRE_BLK = 2000


def _pre2_body(v_ref, ws_ref, we_ref, vws_ref, vwe_ref):
    v = v_ref[...]
    vws_ref[...] = jnp.dot(v, ws_ref[...], preferred_element_type=jnp.float32)
    vwe_ref[...] = jnp.dot(v, we_ref[...], preferred_element_type=jnp.float32)


def _pre1_body(v_ref, w_ref, vw_ref):
    vw_ref[...] = jnp.dot(v_ref[...], w_ref[...],
                          preferred_element_type=jnp.float32)


def _precompute_values(values_table, Ws, We):
    n = values_table.shape[0]
    return pl.pallas_call(
        _pre2_body,
        grid=(n // PRE_BLK,),
        in_specs=[
            pl.BlockSpec((PRE_BLK, D), lambda i: (i, 0)),
            pl.BlockSpec((D, D), lambda i: (0, 0)),
            pl.BlockSpec((D, D), lambda i: (0, 0)),
        ],
        out_specs=[
            pl.BlockSpec((PRE_BLK, D), lambda i: (i, 0)),
            pl.BlockSpec((PRE_BLK, D), lambda i: (i, 0)),
        ],
        out_shape=[
            jax.ShapeDtypeStruct((n, D), jnp.float32),
            jax.ShapeDtypeStruct((n, D), jnp.float32),
        ],
    )(values_table, Ws, We)


def _precompute_paths(paths_table, Wp):
    n = paths_table.shape[0]
    return pl.pallas_call(
        _pre1_body,
        grid=(n // PRE_BLK,),
        in_specs=[
            pl.BlockSpec((PRE_BLK, D), lambda i: (i, 0)),
            pl.BlockSpec((D, D), lambda i: (0, 0)),
        ],
        out_specs=pl.BlockSpec((PRE_BLK, D), lambda i: (i, 0)),
        out_shape=jax.ShapeDtypeStruct((n, D), jnp.float32),
    )(paths_table, Wp)


BB = 16


def _tc_body(s_ref, p_ref, e_ref, st_ref, ws_ref, wp_ref, we_ref,
             a_ref, wo_ref, cv_ref, out_ref):
    s = s_ref[...].reshape(BB * NPATHS, D)
    p = p_ref[...].reshape(BB * NPATHS, D)
    e = e_ref[...].reshape(BB * NPATHS, D)
    acc = jnp.dot(s, ws_ref[...], preferred_element_type=jnp.float32)
    acc = acc + jnp.dot(p, wp_ref[...], preferred_element_type=jnp.float32)
    acc = acc + jnp.dot(e, we_ref[...], preferred_element_type=jnp.float32)
    comb = jnp.tanh(acc)
    a_row = a_ref[...].reshape(1, D)
    logits = jnp.sum(comb * a_row, axis=1).reshape(BB, NPATHS)
    m = (st_ref[...] > 1).astype(jnp.float32)
    z = logits * m + (1.0 - m) * NEG_INF
    zmax = jnp.max(z, axis=1, keepdims=True)
    ez = jnp.exp(z - zmax)
    w = ez / jnp.sum(ez, axis=1, keepdims=True)
    comb3 = comb.reshape(BB, NPATHS, D)
    cv = jnp.sum(comb3 * w[:, :, None], axis=1)
    cv_ref[...] = cv
    out_ref[...] = jnp.dot(cv, wo_ref[...], preferred_element_type=jnp.float32)


def _tc_slim_body(x_ref, st_ref, a_ref, wo_ref, cv_ref, out_ref):
    comb = jnp.tanh(x_ref[...].reshape(BB * NPATHS, D))
    a_row = a_ref[...].reshape(1, D)
    logits = jnp.sum(comb * a_row, axis=1).reshape(BB, NPATHS)
    m = (st_ref[...] > 1).astype(jnp.float32)
    z = logits * m + (1.0 - m) * NEG_INF
    zmax = jnp.max(z, axis=1, keepdims=True)
    ez = jnp.exp(z - zmax)
    w = ez / jnp.sum(ez, axis=1, keepdims=True)
    comb3 = comb.reshape(BB, NPATHS, D)
    cv = jnp.sum(comb3 * w[:, :, None], axis=1)
    cv_ref[...] = cv
    out_ref[...] = jnp.dot(cv, wo_ref[...], preferred_element_type=jnp.float32)


def _tc_slim(chunk, x, starts, a, W_out):
    grid = (BC // BB,)
    boff = chunk * (BC // BB)
    return pl.pallas_call(
        _tc_slim_body,
        grid=grid,
        in_specs=[
            pl.BlockSpec((BB, NPATHS, D), lambda i: (i, 0, 0)),
            pl.BlockSpec((BB, NPATHS), lambda i, boff=boff: (boff + i, 0)),
            pl.BlockSpec((1, D), lambda i: (0, 0)),
            pl.BlockSpec((D, LABELS), lambda i: (0, 0)),
        ],
        out_specs=[
            pl.BlockSpec((BB, D), lambda i: (i, 0)),
            pl.BlockSpec((BB, LABELS), lambda i: (i, 0)),
        ],
        out_shape=[
            jax.ShapeDtypeStruct((BC, D), jnp.float32),
            jax.ShapeDtypeStruct((BC, LABELS), jnp.float32),
        ],
    )(x, starts, a, W_out)


def _tc_dense(chunk, s_g, p_g, e_g, starts, Ws, Wp, We, a, W_out):
    grid = (BC // BB,)
    boff = chunk * (BC // BB)
    return pl.pallas_call(
        _tc_body,
        grid=grid,
        in_specs=[
            pl.BlockSpec((BB, NPATHS, D), lambda i: (i, 0, 0)),
            pl.BlockSpec((BB, NPATHS, D), lambda i: (i, 0, 0)),
            pl.BlockSpec((BB, NPATHS, D), lambda i: (i, 0, 0)),
            pl.BlockSpec((BB, NPATHS), lambda i, boff=boff: (boff + i, 0)),
            pl.BlockSpec((D, D), lambda i: (0, 0)),
            pl.BlockSpec((D, D), lambda i: (0, 0)),
            pl.BlockSpec((D, D), lambda i: (0, 0)),
            pl.BlockSpec((1, D), lambda i: (0, 0)),
            pl.BlockSpec((D, LABELS), lambda i: (0, 0)),
        ],
        out_specs=[
            pl.BlockSpec((BB, D), lambda i: (i, 0)),
            pl.BlockSpec((BB, LABELS), lambda i: (i, 0)),
        ],
        out_shape=[
            jax.ShapeDtypeStruct((BC, D), jnp.float32),
            jax.ShapeDtypeStruct((BC, LABELS), jnp.float32),
        ],
    )(s_g, p_g, e_g, starts, Ws, Wp, We, a, W_out)


def kernel(starts, paths, ends, values_table, paths_table, W, a, W_out):
    starts_f = starts.reshape(B * NPATHS)
    paths_f = paths.reshape(B * NPATHS)
    ends_f = ends.reshape(B * NPATHS)
    Ws, Wp, We = W[:D], W[D:2 * D], W[2 * D:]

    raw_gathered = [
        _sc_gather(c)(starts_f, paths_f, ends_f, values_table, paths_table)
        for c in range(NRAW)]

    VWs, VWe = _precompute_values(values_table, Ws, We)
    PWp = _precompute_paths(paths_table, Wp)

    cvs, outs = [], []
    for c in range(NSPLIT):
        if c < NRAW:
            s_g, p_g, e_g = raw_gathered[c]
            cv, out = _tc_dense(
                c, s_g.reshape(BC, NPATHS, D), p_g.reshape(BC, NPATHS, D),
                e_g.reshape(BC, NPATHS, D), starts, Ws, Wp, We, a, W_out)
        else:
            x = _sc_gsum(c)(starts_f, paths_f, ends_f, VWs, PWp, VWe)
            cv, out = _tc_slim(c, x.reshape(BC, NPATHS, D), starts, a, W_out)
        cvs.append(cv)
        outs.append(out)
    return (jnp.concatenate(cvs, axis=0), jnp.concatenate(outs, axis=0))


# speedup vs baseline: 1.1168x; 1.1168x over previous
import functools

import jax
import jax.numpy as jnp
from jax import lax
from jax.experimental import pallas as pl
from jax.experimental.pallas import tpu as pltpu
from jax.experimental.pallas import tpu_sc as plsc

B = 1024
NPATHS = 200
D = 128
LABELS = 1000
NEG_INF = -2.0 * 10**10

NC = 2
NS = 16
NW = NC * NS

NSPLIT = 4
BC = B // NSPLIT
TOTC = BC * NPATHS
PER_W = TOTC // NW
CHUNK = 80
N_CHUNKS = PER_W // CHUNK


def _sc_gather_body(chunk, starts_h, paths_h, ends_h, vt_h, pt_h,
                    os_h, op_h, oe_h,
                    idx_s, idx_p, idx_e,
                    r00, r01, r02, r10, r11, r12,
                    gsem0, gsem1, ssem0, ssem1):
    wid = lax.axis_index("s") * NC + lax.axis_index("c")
    base = wid * PER_W
    gsem = (gsem0, gsem1)
    ssem = (ssem0, ssem1)
    tabs = (vt_h, pt_h, vt_h)
    outs = (os_h, op_h, oe_h)
    idxs = (idx_s, idx_p, idx_e)
    rows = ((r00, r01, r02), (r10, r11, r12))

    for t, idx_h in enumerate((starts_h, paths_h, ends_h)):
        pltpu.sync_copy(idx_h.at[pl.ds(chunk * TOTC + base, PER_W)], idxs[t])

    def issue_gathers(slot, g):
        for t in range(3):
            pltpu.async_copy(
                tabs[t].at[idxs[t].at[pl.ds(g * CHUNK, CHUNK)]],
                rows[slot][t], gsem[slot])

    def wait_gathers(slot):
        for t in range(3):
            pltpu.make_async_copy(
                tabs[t].at[pl.ds(0, CHUNK), :],
                rows[slot][t], gsem[slot]).wait()

    def issue_stores(slot, g):
        for t in range(3):
            pltpu.async_copy(
                rows[slot][t],
                outs[t].at[pl.ds(base + g * CHUNK, CHUNK), :], ssem[slot])

    def wait_stores(slot):
        for t in range(3):
            pltpu.make_async_copy(
                rows[slot][t],
                outs[t].at[pl.ds(0, CHUNK), :], ssem[slot]).wait()

    issue_gathers(0, 0)
    issue_gathers(1, 1)

    def body(j, _):
        g0 = 2 * j
        g1 = g0 + 1
        wait_gathers(0)
        issue_stores(0, g0)
        wait_gathers(1)
        issue_stores(1, g1)
        wait_stores(0)

        @pl.when(g0 + 2 < N_CHUNKS)
        def _():
            issue_gathers(0, g0 + 2)
        wait_stores(1)

        @pl.when(g1 + 2 < N_CHUNKS)
        def _():
            issue_gathers(1, g1 + 2)
        return 0

    lax.fori_loop(0, N_CHUNKS // 2, body, 0)


@functools.cache
def _sc_gather(chunk):
    return functools.partial(
        pl.kernel,
        mesh=plsc.VectorSubcoreMesh(core_axis_name="c", subcore_axis_name="s"),
        out_type=(
            jax.ShapeDtypeStruct((TOTC, D), jnp.float32),
            jax.ShapeDtypeStruct((TOTC, D), jnp.float32),
            jax.ShapeDtypeStruct((TOTC, D), jnp.float32),
        ),
        scratch_types=(
            [pltpu.VMEM((PER_W,), jnp.int32)] * 3
            + [pltpu.VMEM((CHUNK, D), jnp.float32)] * 6
            + [pltpu.SemaphoreType.DMA] * 4
        ),
    )(functools.partial(_sc_gather_body, chunk))


NRAW = 0


def _sc_gsum_body(chunk, starts_h, paths_h, ends_h, vws_h, pwp_h, vwe_h,
                  out_h,
                  idx_s, idx_p, idx_e,
                  g00, g01, g02, g10, g11, g12, sum0, sum1,
                  gsem0, gsem1, ssem0, ssem1):
    wid = lax.axis_index("s") * NC + lax.axis_index("c")
    base = wid * PER_W
    gsem = (gsem0, gsem1)
    ssem = (ssem0, ssem1)
    tabs = (vws_h, pwp_h, vwe_h)
    idxs = (idx_s, idx_p, idx_e)
    gbuf = ((g00, g01, g02), (g10, g11, g12))
    sbuf = (sum0, sum1)

    for t, idx_h in enumerate((starts_h, paths_h, ends_h)):
        pltpu.sync_copy(idx_h.at[pl.ds(chunk * TOTC + base, PER_W)], idxs[t])

    def issue_gathers(slot, g):
        for t in range(3):
            pltpu.async_copy(
                tabs[t].at[idxs[t].at[pl.ds(g * CHUNK, CHUNK)]],
                gbuf[slot][t], gsem[slot])

    def wait_gathers(slot):
        for t in range(3):
            pltpu.make_async_copy(
                tabs[t].at[pl.ds(0, CHUNK), :],
                gbuf[slot][t], gsem[slot]).wait()

    def compute_sum(slot):
        b0, b1, b2 = gbuf[slot]
        sb = sbuf[slot]

        def row_body(r, _):
            for l in range(D // 16):
                sl = pl.ds(l * 16, 16)
                sb[r, sl] = b0[r, sl] + b1[r, sl] + b2[r, sl]
            return 0
        lax.fori_loop(0, CHUNK, row_body, 0)

    def issue_store(slot, g):
        pltpu.async_copy(
            sbuf[slot], out_h.at[pl.ds(base + g * CHUNK, CHUNK), :],
            ssem[slot])

    def wait_store(slot):
        pltpu.make_async_copy(
            sbuf[slot], out_h.at[pl.ds(0, CHUNK), :], ssem[slot]).wait()

    issue_gathers(0, 0)

    def body(j, _):
        g0 = 2 * j
        g1 = g0 + 1
        wait_gathers(0)
        issue_gathers(1, g1)

        @pl.when(g0 >= 2)
        def _():
            wait_store(0)
        compute_sum(0)
        issue_store(0, g0)
        wait_gathers(1)

        @pl.when(g0 + 2 < N_CHUNKS)
        def _():
            issue_gathers(0, g0 + 2)

        @pl.when(g1 >= 2)
        def _():
            wait_store(1)
        compute_sum(1)
        issue_store(1, g1)
        return 0

    lax.fori_loop(0, N_CHUNKS // 2, body, 0)
    wait_store(0)
    wait_store(1)


@functools.cache
def _sc_gsum(chunk):
    return functools.partial(
        pl.kernel,
        mesh=plsc.VectorSubcoreMesh(core_axis_name="c", subcore_axis_name="s"),
        out_type=jax.ShapeDtypeStruct((TOTC, D), jnp.float32),
        scratch_types=(
            [pltpu.VMEM((PER_W,), jnp.int32)] * 3
            + [pltpu.VMEM((CHUNK, D), jnp.float32)] * 8
            + [pltpu.SemaphoreType.DMA] * 4
        ),
    )(functools.partial(_sc_gsum_body, chunk))


PRE_BLK = 2000


def _pre2_body(v_ref, ws_ref, we_ref, vws_ref, vwe_ref):
    v = v_ref[...]
    vws_ref[...] = jnp.dot(v, ws_ref[...], preferred_element_type=jnp.float32)
    vwe_ref[...] = jnp.dot(v, we_ref[...], preferred_element_type=jnp.float32)


def _pre1_body(v_ref, w_ref, vw_ref):
    vw_ref[...] = jnp.dot(v_ref[...], w_ref[...],
                          preferred_element_type=jnp.float32)


def _precompute_values(values_table, Ws, We):
    n = values_table.shape[0]
    return pl.pallas_call(
        _pre2_body,
        grid=(n // PRE_BLK,),
        in_specs=[
            pl.BlockSpec((PRE_BLK, D), lambda i: (i, 0)),
            pl.BlockSpec((D, D), lambda i: (0, 0)),
            pl.BlockSpec((D, D), lambda i: (0, 0)),
        ],
        out_specs=[
            pl.BlockSpec((PRE_BLK, D), lambda i: (i, 0)),
            pl.BlockSpec((PRE_BLK, D), lambda i: (i, 0)),
        ],
        out_shape=[
            jax.ShapeDtypeStruct((n, D), jnp.float32),
            jax.ShapeDtypeStruct((n, D), jnp.float32),
        ],
    )(values_table, Ws, We)


def _precompute_paths(paths_table, Wp):
    n = paths_table.shape[0]
    return pl.pallas_call(
        _pre1_body,
        grid=(n // PRE_BLK,),
        in_specs=[
            pl.BlockSpec((PRE_BLK, D), lambda i: (i, 0)),
            pl.BlockSpec((D, D), lambda i: (0, 0)),
        ],
        out_specs=pl.BlockSpec((PRE_BLK, D), lambda i: (i, 0)),
        out_shape=jax.ShapeDtypeStruct((n, D), jnp.float32),
    )(paths_table, Wp)


BB = 16


def _tc_body(s_ref, p_ref, e_ref, st_ref, ws_ref, wp_ref, we_ref,
             a_ref, wo_ref, cv_ref, out_ref):
    s = s_ref[...].reshape(BB * NPATHS, D)
    p = p_ref[...].reshape(BB * NPATHS, D)
    e = e_ref[...].reshape(BB * NPATHS, D)
    acc = jnp.dot(s, ws_ref[...], preferred_element_type=jnp.float32)
    acc = acc + jnp.dot(p, wp_ref[...], preferred_element_type=jnp.float32)
    acc = acc + jnp.dot(e, we_ref[...], preferred_element_type=jnp.float32)
    comb = jnp.tanh(acc)
    a_row = a_ref[...].reshape(1, D)
    logits = jnp.sum(comb * a_row, axis=1).reshape(BB, NPATHS)
    m = (st_ref[...] > 1).astype(jnp.float32)
    z = logits * m + (1.0 - m) * NEG_INF
    zmax = jnp.max(z, axis=1, keepdims=True)
    ez = jnp.exp(z - zmax)
    w = ez / jnp.sum(ez, axis=1, keepdims=True)
    comb3 = comb.reshape(BB, NPATHS, D)
    cv = jnp.sum(comb3 * w[:, :, None], axis=1)
    cv_ref[...] = cv
    out_ref[...] = jnp.dot(cv, wo_ref[...], preferred_element_type=jnp.float32)


def _tc_slim_body(x_ref, st_ref, a_ref, wo_ref, cv_ref, out_ref):
    comb = jnp.tanh(x_ref[...].reshape(BB * NPATHS, D))
    a_row = a_ref[...].reshape(1, D)
    logits = jnp.sum(comb * a_row, axis=1).reshape(BB, NPATHS)
    m = (st_ref[...] > 1).astype(jnp.float32)
    z = logits * m + (1.0 - m) * NEG_INF
    zmax = jnp.max(z, axis=1, keepdims=True)
    ez = jnp.exp(z - zmax)
    w = ez / jnp.sum(ez, axis=1, keepdims=True)
    comb3 = comb.reshape(BB, NPATHS, D)
    cv = jnp.sum(comb3 * w[:, :, None], axis=1)
    cv_ref[...] = cv
    out_ref[...] = jnp.dot(cv, wo_ref[...], preferred_element_type=jnp.float32)


def _tc_slim(chunk, x, starts, a, W_out):
    grid = (BC // BB,)
    boff = chunk * (BC // BB)
    return pl.pallas_call(
        _tc_slim_body,
        grid=grid,
        in_specs=[
            pl.BlockSpec((BB, NPATHS, D), lambda i: (i, 0, 0)),
            pl.BlockSpec((BB, NPATHS), lambda i, boff=boff: (boff + i, 0)),
            pl.BlockSpec((1, D), lambda i: (0, 0)),
            pl.BlockSpec((D, LABELS), lambda i: (0, 0)),
        ],
        out_specs=[
            pl.BlockSpec((BB, D), lambda i: (i, 0)),
            pl.BlockSpec((BB, LABELS), lambda i: (i, 0)),
        ],
        out_shape=[
            jax.ShapeDtypeStruct((BC, D), jnp.float32),
            jax.ShapeDtypeStruct((BC, LABELS), jnp.float32),
        ],
    )(x, starts, a, W_out)


def _tc_dense(chunk, s_g, p_g, e_g, starts, Ws, Wp, We, a, W_out):
    grid = (BC // BB,)
    boff = chunk * (BC // BB)
    return pl.pallas_call(
        _tc_body,
        grid=grid,
        in_specs=[
            pl.BlockSpec((BB, NPATHS, D), lambda i: (i, 0, 0)),
            pl.BlockSpec((BB, NPATHS, D), lambda i: (i, 0, 0)),
            pl.BlockSpec((BB, NPATHS, D), lambda i: (i, 0, 0)),
            pl.BlockSpec((BB, NPATHS), lambda i, boff=boff: (boff + i, 0)),
            pl.BlockSpec((D, D), lambda i: (0, 0)),
            pl.BlockSpec((D, D), lambda i: (0, 0)),
            pl.BlockSpec((D, D), lambda i: (0, 0)),
            pl.BlockSpec((1, D), lambda i: (0, 0)),
            pl.BlockSpec((D, LABELS), lambda i: (0, 0)),
        ],
        out_specs=[
            pl.BlockSpec((BB, D), lambda i: (i, 0)),
            pl.BlockSpec((BB, LABELS), lambda i: (i, 0)),
        ],
        out_shape=[
            jax.ShapeDtypeStruct((BC, D), jnp.float32),
            jax.ShapeDtypeStruct((BC, LABELS), jnp.float32),
        ],
    )(s_g, p_g, e_g, starts, Ws, Wp, We, a, W_out)


def kernel(starts, paths, ends, values_table, paths_table, W, a, W_out):
    starts_f = starts.reshape(B * NPATHS)
    paths_f = paths.reshape(B * NPATHS)
    ends_f = ends.reshape(B * NPATHS)
    Ws, Wp, We = W[:D], W[D:2 * D], W[2 * D:]

    raw_gathered = [
        _sc_gather(c)(starts_f, paths_f, ends_f, values_table, paths_table)
        for c in range(NRAW)]

    VWs, VWe = _precompute_values(values_table, Ws, We)
    PWp = _precompute_paths(paths_table, Wp)

    cvs, outs = [], []
    for c in range(NSPLIT):
        if c < NRAW:
            s_g, p_g, e_g = raw_gathered[c]
            cv, out = _tc_dense(
                c, s_g.reshape(BC, NPATHS, D), p_g.reshape(BC, NPATHS, D),
                e_g.reshape(BC, NPATHS, D), starts, Ws, Wp, We, a, W_out)
        else:
            x = _sc_gsum(c)(starts_f, paths_f, ends_f, VWs, PWp, VWe)
            cv, out = _tc_slim(c, x.reshape(BC, NPATHS, D), starts, a, W_out)
        cvs.append(cv)
        outs.append(out)
    return (jnp.concatenate(cvs, axis=0), jnp.concatenate(outs, axis=0))


# speedup vs baseline: 1.2368x; 1.1074x over previous
import functools

import jax
import jax.numpy as jnp
from jax import lax
from jax.experimental import pallas as pl
from jax.experimental.pallas import tpu as pltpu
from jax.experimental.pallas import tpu_sc as plsc

B = 1024
NPATHS = 200
D = 128
LABELS = 1000
NEG_INF = -2.0 * 10**10

NC = 2
NS = 16
NW = NC * NS

NSPLIT = 4
BC = B // NSPLIT
TOTC = BC * NPATHS
PER_W = TOTC // NW
CHUNK = 80
N_CHUNKS = PER_W // CHUNK


def _sc_gather_body(chunk, starts_h, paths_h, ends_h, vt_h, pt_h,
                    os_h, op_h, oe_h,
                    idx_s, idx_p, idx_e,
                    r00, r01, r02, r10, r11, r12,
                    gsem0, gsem1, ssem0, ssem1):
    wid = lax.axis_index("s") * NC + lax.axis_index("c")
    base = wid * PER_W
    gsem = (gsem0, gsem1)
    ssem = (ssem0, ssem1)
    tabs = (vt_h, pt_h, vt_h)
    outs = (os_h, op_h, oe_h)
    idxs = (idx_s, idx_p, idx_e)
    rows = ((r00, r01, r02), (r10, r11, r12))

    for t, idx_h in enumerate((starts_h, paths_h, ends_h)):
        pltpu.sync_copy(idx_h.at[pl.ds(chunk * TOTC + base, PER_W)], idxs[t])

    def issue_gathers(slot, g):
        for t in range(3):
            pltpu.async_copy(
                tabs[t].at[idxs[t].at[pl.ds(g * CHUNK, CHUNK)]],
                rows[slot][t], gsem[slot])

    def wait_gathers(slot):
        for t in range(3):
            pltpu.make_async_copy(
                tabs[t].at[pl.ds(0, CHUNK), :],
                rows[slot][t], gsem[slot]).wait()

    def issue_stores(slot, g):
        for t in range(3):
            pltpu.async_copy(
                rows[slot][t],
                outs[t].at[pl.ds(base + g * CHUNK, CHUNK), :], ssem[slot])

    def wait_stores(slot):
        for t in range(3):
            pltpu.make_async_copy(
                rows[slot][t],
                outs[t].at[pl.ds(0, CHUNK), :], ssem[slot]).wait()

    issue_gathers(0, 0)
    issue_gathers(1, 1)

    def body(j, _):
        g0 = 2 * j
        g1 = g0 + 1
        wait_gathers(0)
        issue_stores(0, g0)
        wait_gathers(1)
        issue_stores(1, g1)
        wait_stores(0)

        @pl.when(g0 + 2 < N_CHUNKS)
        def _():
            issue_gathers(0, g0 + 2)
        wait_stores(1)

        @pl.when(g1 + 2 < N_CHUNKS)
        def _():
            issue_gathers(1, g1 + 2)
        return 0

    lax.fori_loop(0, N_CHUNKS // 2, body, 0)


@functools.cache
def _sc_gather(chunk):
    return functools.partial(
        pl.kernel,
        mesh=plsc.VectorSubcoreMesh(core_axis_name="c", subcore_axis_name="s"),
        out_type=(
            jax.ShapeDtypeStruct((TOTC, D), jnp.float32),
            jax.ShapeDtypeStruct((TOTC, D), jnp.float32),
            jax.ShapeDtypeStruct((TOTC, D), jnp.float32),
        ),
        scratch_types=(
            [pltpu.VMEM((PER_W,), jnp.int32)] * 3
            + [pltpu.VMEM((CHUNK, D), jnp.float32)] * 6
            + [pltpu.SemaphoreType.DMA] * 4
        ),
    )(functools.partial(_sc_gather_body, chunk))


NRAW = 0


def _sc_gsum_body(chunk, starts_h, paths_h, ends_h, vws_h, pwp_h, vwe_h,
                  out_h,
                  idx_s, idx_p, idx_e,
                  g00, g01, g02, g10, g11, g12, sum0, sum1,
                  gsem0, gsem1, ssem0, ssem1):
    wid = lax.axis_index("s") * NC + lax.axis_index("c")
    base = wid * PER_W
    gsem = (gsem0, gsem1)
    ssem = (ssem0, ssem1)
    tabs = (vws_h, pwp_h, vwe_h)
    idxs = (idx_s, idx_p, idx_e)
    gbuf = ((g00, g01, g02), (g10, g11, g12))
    sbuf = (sum0, sum1)

    for t, idx_h in enumerate((starts_h, paths_h, ends_h)):
        pltpu.sync_copy(idx_h.at[pl.ds(chunk * TOTC + base, PER_W)], idxs[t])

    def issue_gathers(slot, g):
        for t in range(3):
            pltpu.async_copy(
                tabs[t].at[idxs[t].at[pl.ds(g * CHUNK, CHUNK)]],
                gbuf[slot][t], gsem[slot])

    def wait_gathers(slot):
        for t in range(3):
            pltpu.make_async_copy(
                tabs[t].at[pl.ds(0, CHUNK), :],
                gbuf[slot][t], gsem[slot]).wait()

    def compute_sum(slot):
        b0, b1, b2 = gbuf[slot]
        sb = sbuf[slot]

        def row_body(r, _):
            for l in range(D // 16):
                sl = pl.ds(l * 16, 16)
                sb[r, sl] = b0[r, sl] + b1[r, sl] + b2[r, sl]
            return 0
        lax.fori_loop(0, CHUNK, row_body, 0)

    def issue_store(slot, g):
        pltpu.async_copy(
            sbuf[slot], out_h.at[pl.ds(base + g * CHUNK, CHUNK), :],
            ssem[slot])

    def wait_store(slot):
        pltpu.make_async_copy(
            sbuf[slot], out_h.at[pl.ds(0, CHUNK), :], ssem[slot]).wait()

    issue_gathers(0, 0)

    def body(j, _):
        g0 = 2 * j
        g1 = g0 + 1
        wait_gathers(0)
        issue_gathers(1, g1)

        @pl.when(g0 >= 2)
        def _():
            wait_store(0)
        compute_sum(0)
        issue_store(0, g0)
        wait_gathers(1)

        @pl.when(g0 + 2 < N_CHUNKS)
        def _():
            issue_gathers(0, g0 + 2)

        @pl.when(g1 >= 2)
        def _():
            wait_store(1)
        compute_sum(1)
        issue_store(1, g1)
        return 0

    lax.fori_loop(0, N_CHUNKS // 2, body, 0)
    wait_store(0)
    wait_store(1)


@functools.cache
def _sc_gsum(chunk):
    return functools.partial(
        pl.kernel,
        mesh=plsc.VectorSubcoreMesh(core_axis_name="c", subcore_axis_name="s"),
        out_type=jax.ShapeDtypeStruct((TOTC, D), jnp.float32),
        scratch_types=(
            [pltpu.VMEM((PER_W,), jnp.int32)] * 3
            + [pltpu.VMEM((CHUNK, D), jnp.float32)] * 8
            + [pltpu.SemaphoreType.DMA] * 4
        ),
    )(functools.partial(_sc_gsum_body, chunk))


PRE_BLK = 2000


def _pre2_body(v_ref, ws_ref, we_ref, vws_ref, vwe_ref):
    v = v_ref[...]
    vws_ref[...] = jnp.dot(v, ws_ref[...], preferred_element_type=jnp.float32)
    vwe_ref[...] = jnp.dot(v, we_ref[...], preferred_element_type=jnp.float32)


def _pre1_body(v_ref, w_ref, vw_ref):
    vw_ref[...] = jnp.dot(v_ref[...], w_ref[...],
                          preferred_element_type=jnp.float32)


def _precompute_values(values_table, Ws, We):
    n = values_table.shape[0]
    return pl.pallas_call(
        _pre2_body,
        grid=(n // PRE_BLK,),
        in_specs=[
            pl.BlockSpec((PRE_BLK, D), lambda i: (i, 0)),
            pl.BlockSpec((D, D), lambda i: (0, 0)),
            pl.BlockSpec((D, D), lambda i: (0, 0)),
        ],
        out_specs=[
            pl.BlockSpec((PRE_BLK, D), lambda i: (i, 0)),
            pl.BlockSpec((PRE_BLK, D), lambda i: (i, 0)),
        ],
        out_shape=[
            jax.ShapeDtypeStruct((n, D), jnp.float32),
            jax.ShapeDtypeStruct((n, D), jnp.float32),
        ],
    )(values_table, Ws, We)


def _precompute_paths(paths_table, Wp):
    n = paths_table.shape[0]
    return pl.pallas_call(
        _pre1_body,
        grid=(n // PRE_BLK,),
        in_specs=[
            pl.BlockSpec((PRE_BLK, D), lambda i: (i, 0)),
            pl.BlockSpec((D, D), lambda i: (0, 0)),
        ],
        out_specs=pl.BlockSpec((PRE_BLK, D), lambda i: (i, 0)),
        out_shape=jax.ShapeDtypeStruct((n, D), jnp.float32),
    )(paths_table, Wp)


BB = 32


def _tc_body(s_ref, p_ref, e_ref, st_ref, ws_ref, wp_ref, we_ref,
             a_ref, wo_ref, cv_ref, out_ref):
    s = s_ref[...].reshape(BB * NPATHS, D)
    p = p_ref[...].reshape(BB * NPATHS, D)
    e = e_ref[...].reshape(BB * NPATHS, D)
    acc = jnp.dot(s, ws_ref[...], preferred_element_type=jnp.float32)
    acc = acc + jnp.dot(p, wp_ref[...], preferred_element_type=jnp.float32)
    acc = acc + jnp.dot(e, we_ref[...], preferred_element_type=jnp.float32)
    comb = jnp.tanh(acc)
    a_row = a_ref[...].reshape(1, D)
    logits = jnp.sum(comb * a_row, axis=1).reshape(BB, NPATHS)
    m = (st_ref[...] > 1).astype(jnp.float32)
    z = logits * m + (1.0 - m) * NEG_INF
    zmax = jnp.max(z, axis=1, keepdims=True)
    ez = jnp.exp(z - zmax)
    w = ez / jnp.sum(ez, axis=1, keepdims=True)
    comb3 = comb.reshape(BB, NPATHS, D)
    cv = jnp.sum(comb3 * w[:, :, None], axis=1)
    cv_ref[...] = cv
    out_ref[...] = jnp.dot(cv, wo_ref[...], preferred_element_type=jnp.float32)


def _tc_slim_body(x_ref, st_ref, a_ref, wo_ref, cv_ref, out_ref):
    comb = jnp.tanh(x_ref[...])
    a_row = a_ref[...].reshape(1, 1, D)
    logits = jnp.sum(comb * a_row, axis=2)
    z = jnp.where(st_ref[...] > 1, logits, NEG_INF)
    zmax = jnp.max(z, axis=1, keepdims=True)
    ez = jnp.exp(z - zmax)
    w = ez / jnp.sum(ez, axis=1, keepdims=True)
    cv = jnp.sum(comb * w[:, :, None], axis=1)
    cv_ref[...] = cv
    out_ref[...] = jnp.dot(cv, wo_ref[...], preferred_element_type=jnp.float32)


def _tc_slim(chunk, x, starts, a, W_out):
    grid = (BC // BB,)
    boff = chunk * (BC // BB)
    return pl.pallas_call(
        _tc_slim_body,
        grid=grid,
        in_specs=[
            pl.BlockSpec((BB, NPATHS, D), lambda i: (i, 0, 0)),
            pl.BlockSpec((BB, NPATHS), lambda i, boff=boff: (boff + i, 0)),
            pl.BlockSpec((1, D), lambda i: (0, 0)),
            pl.BlockSpec((D, LABELS), lambda i: (0, 0)),
        ],
        out_specs=[
            pl.BlockSpec((BB, D), lambda i: (i, 0)),
            pl.BlockSpec((BB, LABELS), lambda i: (i, 0)),
        ],
        out_shape=[
            jax.ShapeDtypeStruct((BC, D), jnp.float32),
            jax.ShapeDtypeStruct((BC, LABELS), jnp.float32),
        ],
    )(x, starts, a, W_out)


def _tc_dense(chunk, s_g, p_g, e_g, starts, Ws, Wp, We, a, W_out):
    grid = (BC // BB,)
    boff = chunk * (BC // BB)
    return pl.pallas_call(
        _tc_body,
        grid=grid,
        in_specs=[
            pl.BlockSpec((BB, NPATHS, D), lambda i: (i, 0, 0)),
            pl.BlockSpec((BB, NPATHS, D), lambda i: (i, 0, 0)),
            pl.BlockSpec((BB, NPATHS, D), lambda i: (i, 0, 0)),
            pl.BlockSpec((BB, NPATHS), lambda i, boff=boff: (boff + i, 0)),
            pl.BlockSpec((D, D), lambda i: (0, 0)),
            pl.BlockSpec((D, D), lambda i: (0, 0)),
            pl.BlockSpec((D, D), lambda i: (0, 0)),
            pl.BlockSpec((1, D), lambda i: (0, 0)),
            pl.BlockSpec((D, LABELS), lambda i: (0, 0)),
        ],
        out_specs=[
            pl.BlockSpec((BB, D), lambda i: (i, 0)),
            pl.BlockSpec((BB, LABELS), lambda i: (i, 0)),
        ],
        out_shape=[
            jax.ShapeDtypeStruct((BC, D), jnp.float32),
            jax.ShapeDtypeStruct((BC, LABELS), jnp.float32),
        ],
    )(s_g, p_g, e_g, starts, Ws, Wp, We, a, W_out)


def kernel(starts, paths, ends, values_table, paths_table, W, a, W_out):
    starts_f = starts.reshape(B * NPATHS)
    paths_f = paths.reshape(B * NPATHS)
    ends_f = ends.reshape(B * NPATHS)
    Ws, Wp, We = W[:D], W[D:2 * D], W[2 * D:]

    raw_gathered = [
        _sc_gather(c)(starts_f, paths_f, ends_f, values_table, paths_table)
        for c in range(NRAW)]

    VWs, VWe = _precompute_values(values_table, Ws, We)
    PWp = _precompute_paths(paths_table, Wp)

    cvs, outs = [], []
    for c in range(NSPLIT):
        if c < NRAW:
            s_g, p_g, e_g = raw_gathered[c]
            cv, out = _tc_dense(
                c, s_g.reshape(BC, NPATHS, D), p_g.reshape(BC, NPATHS, D),
                e_g.reshape(BC, NPATHS, D), starts, Ws, Wp, We, a, W_out)
        else:
            x = _sc_gsum(c)(starts_f, paths_f, ends_f, VWs, PWp, VWe)
            cv, out = _tc_slim(c, x.reshape(BC, NPATHS, D), starts, a, W_out)
        cvs.append(cv)
        outs.append(out)
    return (jnp.concatenate(cvs, axis=0), jnp.concatenate(outs, axis=0))


# speedup vs baseline: 1.2595x; 1.0183x over previous
import functools

import jax
import jax.numpy as jnp
from jax import lax
from jax.experimental import pallas as pl
from jax.experimental.pallas import tpu as pltpu
from jax.experimental.pallas import tpu_sc as plsc

B = 1024
NPATHS = 200
D = 128
LABELS = 1000
NEG_INF = -2.0 * 10**10

NC = 2
NS = 16
NW = NC * NS

NSPLIT = 4
BC = B // NSPLIT
TOTC = BC * NPATHS
PER_W = TOTC // NW
CHUNK = 80
N_CHUNKS = PER_W // CHUNK


def _sc_gather_body(chunk, starts_h, paths_h, ends_h, vt_h, pt_h,
                    os_h, op_h, oe_h,
                    idx_s, idx_p, idx_e,
                    r00, r01, r02, r10, r11, r12,
                    gsem0, gsem1, ssem0, ssem1):
    wid = lax.axis_index("s") * NC + lax.axis_index("c")
    base = wid * PER_W
    gsem = (gsem0, gsem1)
    ssem = (ssem0, ssem1)
    tabs = (vt_h, pt_h, vt_h)
    outs = (os_h, op_h, oe_h)
    idxs = (idx_s, idx_p, idx_e)
    rows = ((r00, r01, r02), (r10, r11, r12))

    for t, idx_h in enumerate((starts_h, paths_h, ends_h)):
        pltpu.sync_copy(idx_h.at[pl.ds(chunk * TOTC + base, PER_W)], idxs[t])

    def issue_gathers(slot, g):
        for t in range(3):
            pltpu.async_copy(
                tabs[t].at[idxs[t].at[pl.ds(g * CHUNK, CHUNK)]],
                rows[slot][t], gsem[slot])

    def wait_gathers(slot):
        for t in range(3):
            pltpu.make_async_copy(
                tabs[t].at[pl.ds(0, CHUNK), :],
                rows[slot][t], gsem[slot]).wait()

    def issue_stores(slot, g):
        for t in range(3):
            pltpu.async_copy(
                rows[slot][t],
                outs[t].at[pl.ds(base + g * CHUNK, CHUNK), :], ssem[slot])

    def wait_stores(slot):
        for t in range(3):
            pltpu.make_async_copy(
                rows[slot][t],
                outs[t].at[pl.ds(0, CHUNK), :], ssem[slot]).wait()

    issue_gathers(0, 0)
    issue_gathers(1, 1)

    def body(j, _):
        g0 = 2 * j
        g1 = g0 + 1
        wait_gathers(0)
        issue_stores(0, g0)
        wait_gathers(1)
        issue_stores(1, g1)
        wait_stores(0)

        @pl.when(g0 + 2 < N_CHUNKS)
        def _():
            issue_gathers(0, g0 + 2)
        wait_stores(1)

        @pl.when(g1 + 2 < N_CHUNKS)
        def _():
            issue_gathers(1, g1 + 2)
        return 0

    lax.fori_loop(0, N_CHUNKS // 2, body, 0)


@functools.cache
def _sc_gather(chunk):
    return functools.partial(
        pl.kernel,
        mesh=plsc.VectorSubcoreMesh(core_axis_name="c", subcore_axis_name="s"),
        out_type=(
            jax.ShapeDtypeStruct((TOTC, D), jnp.float32),
            jax.ShapeDtypeStruct((TOTC, D), jnp.float32),
            jax.ShapeDtypeStruct((TOTC, D), jnp.float32),
        ),
        scratch_types=(
            [pltpu.VMEM((PER_W,), jnp.int32)] * 3
            + [pltpu.VMEM((CHUNK, D), jnp.float32)] * 6
            + [pltpu.SemaphoreType.DMA] * 4
        ),
    )(functools.partial(_sc_gather_body, chunk))


NRAW = 0


def _sc_gsum_body(chunk, starts_h, paths_h, ends_h, vws_h, pwp_h, vwe_h,
                  out_h,
                  idx_s, idx_p, idx_e,
                  sum0, sum1,
                  gsem0, gsem1, ssem0, ssem1):
    wid = lax.axis_index("s") * NC + lax.axis_index("c")
    base = wid * PER_W
    gsem = (gsem0, gsem1)
    ssem = (ssem0, ssem1)
    tabs = (vws_h, pwp_h, vwe_h)
    idxs = (idx_s, idx_p, idx_e)
    sbuf = (sum0, sum1)

    for t, idx_h in enumerate((starts_h, paths_h, ends_h)):
        pltpu.sync_copy(idx_h.at[pl.ds(chunk * TOTC + base, PER_W)], idxs[t])

    zvec = jnp.zeros((16,), jnp.float32)

    def zero_sum(slot):
        sb = sbuf[slot]

        def row_body(r, _):
            for l in range(D // 16):
                sb[r, pl.ds(l * 16, 16)] = zvec
            return 0
        lax.fori_loop(0, CHUNK, row_body, 0)

    def issue_gathers(slot, g):
        for t in range(3):
            pltpu.async_copy(
                tabs[t].at[idxs[t].at[pl.ds(g * CHUNK, CHUNK)]],
                sbuf[slot], gsem[slot], add=True)

    def wait_gathers(slot):
        for t in range(3):
            pltpu.make_async_copy(
                tabs[t].at[pl.ds(0, CHUNK), :],
                sbuf[slot], gsem[slot]).wait()

    def issue_store(slot, g):
        pltpu.async_copy(
            sbuf[slot], out_h.at[pl.ds(base + g * CHUNK, CHUNK), :],
            ssem[slot])

    def wait_store(slot):
        pltpu.make_async_copy(
            sbuf[slot], out_h.at[pl.ds(0, CHUNK), :], ssem[slot]).wait()

    zero_sum(0)
    issue_gathers(0, 0)
    zero_sum(1)
    issue_gathers(1, 1)

    def body(j, _):
        g0 = 2 * j
        g1 = g0 + 1
        wait_gathers(0)
        issue_store(0, g0)
        wait_gathers(1)
        issue_store(1, g1)

        @pl.when(g0 + 2 < N_CHUNKS)
        def _():
            wait_store(0)
            zero_sum(0)
            issue_gathers(0, g0 + 2)

        @pl.when(g1 + 2 < N_CHUNKS)
        def _():
            wait_store(1)
            zero_sum(1)
            issue_gathers(1, g1 + 2)
        return 0

    lax.fori_loop(0, N_CHUNKS // 2, body, 0)
    wait_store(0)
    wait_store(1)


@functools.cache
def _sc_gsum(chunk):
    return functools.partial(
        pl.kernel,
        mesh=plsc.VectorSubcoreMesh(core_axis_name="c", subcore_axis_name="s"),
        out_type=jax.ShapeDtypeStruct((TOTC, D), jnp.float32),
        scratch_types=(
            [pltpu.VMEM((PER_W,), jnp.int32)] * 3
            + [pltpu.VMEM((CHUNK, D), jnp.float32)] * 2
            + [pltpu.SemaphoreType.DMA] * 4
        ),
    )(functools.partial(_sc_gsum_body, chunk))


PRE_BLK = 2000


def _pre2_body(v_ref, ws_ref, we_ref, vws_ref, vwe_ref):
    v = v_ref[...]
    vws_ref[...] = jnp.dot(v, ws_ref[...], preferred_element_type=jnp.float32)
    vwe_ref[...] = jnp.dot(v, we_ref[...], preferred_element_type=jnp.float32)


def _pre1_body(v_ref, w_ref, vw_ref):
    vw_ref[...] = jnp.dot(v_ref[...], w_ref[...],
                          preferred_element_type=jnp.float32)


def _precompute_values(values_table, Ws, We):
    n = values_table.shape[0]
    return pl.pallas_call(
        _pre2_body,
        grid=(n // PRE_BLK,),
        in_specs=[
            pl.BlockSpec((PRE_BLK, D), lambda i: (i, 0)),
            pl.BlockSpec((D, D), lambda i: (0, 0)),
            pl.BlockSpec((D, D), lambda i: (0, 0)),
        ],
        out_specs=[
            pl.BlockSpec((PRE_BLK, D), lambda i: (i, 0)),
            pl.BlockSpec((PRE_BLK, D), lambda i: (i, 0)),
        ],
        out_shape=[
            jax.ShapeDtypeStruct((n, D), jnp.float32),
            jax.ShapeDtypeStruct((n, D), jnp.float32),
        ],
    )(values_table, Ws, We)


def _precompute_paths(paths_table, Wp):
    n = paths_table.shape[0]
    return pl.pallas_call(
        _pre1_body,
        grid=(n // PRE_BLK,),
        in_specs=[
            pl.BlockSpec((PRE_BLK, D), lambda i: (i, 0)),
            pl.BlockSpec((D, D), lambda i: (0, 0)),
        ],
        out_specs=pl.BlockSpec((PRE_BLK, D), lambda i: (i, 0)),
        out_shape=jax.ShapeDtypeStruct((n, D), jnp.float32),
    )(paths_table, Wp)


BB = 32


def _tc_body(s_ref, p_ref, e_ref, st_ref, ws_ref, wp_ref, we_ref,
             a_ref, wo_ref, cv_ref, out_ref):
    s = s_ref[...].reshape(BB * NPATHS, D)
    p = p_ref[...].reshape(BB * NPATHS, D)
    e = e_ref[...].reshape(BB * NPATHS, D)
    acc = jnp.dot(s, ws_ref[...], preferred_element_type=jnp.float32)
    acc = acc + jnp.dot(p, wp_ref[...], preferred_element_type=jnp.float32)
    acc = acc + jnp.dot(e, we_ref[...], preferred_element_type=jnp.float32)
    comb = jnp.tanh(acc)
    a_row = a_ref[...].reshape(1, D)
    logits = jnp.sum(comb * a_row, axis=1).reshape(BB, NPATHS)
    m = (st_ref[...] > 1).astype(jnp.float32)
    z = logits * m + (1.0 - m) * NEG_INF
    zmax = jnp.max(z, axis=1, keepdims=True)
    ez = jnp.exp(z - zmax)
    w = ez / jnp.sum(ez, axis=1, keepdims=True)
    comb3 = comb.reshape(BB, NPATHS, D)
    cv = jnp.sum(comb3 * w[:, :, None], axis=1)
    cv_ref[...] = cv
    out_ref[...] = jnp.dot(cv, wo_ref[...], preferred_element_type=jnp.float32)


def _tc_slim_body(x_ref, st_ref, a_ref, wo_ref, cv_ref, out_ref):
    comb = jnp.tanh(x_ref[...])
    a_row = a_ref[...].reshape(1, 1, D)
    logits = jnp.sum(comb * a_row, axis=2)
    z = jnp.where(st_ref[...] > 1, logits, NEG_INF)
    zmax = jnp.max(z, axis=1, keepdims=True)
    ez = jnp.exp(z - zmax)
    w = ez / jnp.sum(ez, axis=1, keepdims=True)
    cv = jnp.sum(comb * w[:, :, None], axis=1)
    cv_ref[...] = cv
    out_ref[...] = jnp.dot(cv, wo_ref[...], preferred_element_type=jnp.float32)


def _tc_slim(chunk, x, starts, a, W_out):
    grid = (BC // BB,)
    boff = chunk * (BC // BB)
    return pl.pallas_call(
        _tc_slim_body,
        grid=grid,
        in_specs=[
            pl.BlockSpec((BB, NPATHS, D), lambda i: (i, 0, 0)),
            pl.BlockSpec((BB, NPATHS), lambda i, boff=boff: (boff + i, 0)),
            pl.BlockSpec((1, D), lambda i: (0, 0)),
            pl.BlockSpec((D, LABELS), lambda i: (0, 0)),
        ],
        out_specs=[
            pl.BlockSpec((BB, D), lambda i: (i, 0)),
            pl.BlockSpec((BB, LABELS), lambda i: (i, 0)),
        ],
        out_shape=[
            jax.ShapeDtypeStruct((BC, D), jnp.float32),
            jax.ShapeDtypeStruct((BC, LABELS), jnp.float32),
        ],
    )(x, starts, a, W_out)


def _tc_dense(chunk, s_g, p_g, e_g, starts, Ws, Wp, We, a, W_out):
    grid = (BC // BB,)
    boff = chunk * (BC // BB)
    return pl.pallas_call(
        _tc_body,
        grid=grid,
        in_specs=[
            pl.BlockSpec((BB, NPATHS, D), lambda i: (i, 0, 0)),
            pl.BlockSpec((BB, NPATHS, D), lambda i: (i, 0, 0)),
            pl.BlockSpec((BB, NPATHS, D), lambda i: (i, 0, 0)),
            pl.BlockSpec((BB, NPATHS), lambda i, boff=boff: (boff + i, 0)),
            pl.BlockSpec((D, D), lambda i: (0, 0)),
            pl.BlockSpec((D, D), lambda i: (0, 0)),
            pl.BlockSpec((D, D), lambda i: (0, 0)),
            pl.BlockSpec((1, D), lambda i: (0, 0)),
            pl.BlockSpec((D, LABELS), lambda i: (0, 0)),
        ],
        out_specs=[
            pl.BlockSpec((BB, D), lambda i: (i, 0)),
            pl.BlockSpec((BB, LABELS), lambda i: (i, 0)),
        ],
        out_shape=[
            jax.ShapeDtypeStruct((BC, D), jnp.float32),
            jax.ShapeDtypeStruct((BC, LABELS), jnp.float32),
        ],
    )(s_g, p_g, e_g, starts, Ws, Wp, We, a, W_out)


def kernel(starts, paths, ends, values_table, paths_table, W, a, W_out):
    starts_f = starts.reshape(B * NPATHS)
    paths_f = paths.reshape(B * NPATHS)
    ends_f = ends.reshape(B * NPATHS)
    Ws, Wp, We = W[:D], W[D:2 * D], W[2 * D:]

    raw_gathered = [
        _sc_gather(c)(starts_f, paths_f, ends_f, values_table, paths_table)
        for c in range(NRAW)]

    VWs, VWe = _precompute_values(values_table, Ws, We)
    PWp = _precompute_paths(paths_table, Wp)

    cvs, outs = [], []
    for c in range(NSPLIT):
        if c < NRAW:
            s_g, p_g, e_g = raw_gathered[c]
            cv, out = _tc_dense(
                c, s_g.reshape(BC, NPATHS, D), p_g.reshape(BC, NPATHS, D),
                e_g.reshape(BC, NPATHS, D), starts, Ws, Wp, We, a, W_out)
        else:
            x = _sc_gsum(c)(starts_f, paths_f, ends_f, VWs, PWp, VWe)
            cv, out = _tc_slim(c, x.reshape(BC, NPATHS, D), starts, a, W_out)
        cvs.append(cv)
        outs.append(out)
    return (jnp.concatenate(cvs, axis=0), jnp.concatenate(outs, axis=0))
